# baseline (speedup 1.0000x reference)
import functools

import jax
import jax.numpy as jnp
from jax import lax
from jax.experimental import pallas as pl
from jax.experimental.pallas import tpu as pltpu
from jax.experimental.pallas import tpu_sc as plsc

B = 8
HW = 1024
N = B * HW
D = 512
MS = 2048
TN = 1024
NC, NS = 2, 16
NEG = -3.0e38
LANES = 16

DS = D // NS
PSC = N // NC
CHP = 512
NCHK = PSC // CHP
ACC_N = MS * DS


def _tc1_body(q_ref, mem_ref, memb_ref, qh_ref, a_ref, sval_ref, m_ref):
    i = pl.program_id(0)
    q = q_ref[...]
    score = lax.dot_general(q, mem_ref[...], (((1,), (1,)), ((), ())),
                            preferred_element_type=jnp.float32)
    rowmax = jnp.max(score, axis=1, keepdims=True)
    e = jnp.exp(score - rowmax)
    z = jnp.sum(e, axis=1, keepdims=True)
    eqf = jnp.where(score == rowmax, 1.0, 0.0)
    iota_col = lax.broadcasted_iota(jnp.int32, (MS, 1), 0).astype(jnp.float32)
    a = lax.dot_general(eqf, iota_col, (((1,), (0,)), ((), ())),
                        preferred_element_type=jnp.float32)
    s_e = jnp.where(e < z * (1.0 / MS), 0.0, e)
    denom = jnp.maximum(jnp.sum(s_e, axis=1, keepdims=True), 1e-12)
    qh = lax.dot_general(s_e.astype(jnp.bfloat16), memb_ref[...],
                         (((1,), (0,)), ((), ())),
                         preferred_element_type=jnp.float32)
    qh_ref[...] = qh * (1.0 / denom)
    a_ref[...] = a[:, 0].astype(jnp.int32)
    sval_ref[...] = rowmax[:, 0]
    mpart = jnp.max(jnp.where(eqf > 0.0, rowmax, NEG), axis=0)

    @pl.when(i == 0)
    def _():
        m_ref[...] = mpart

    @pl.when(i != 0)
    def _():
        m_ref[...] = jnp.maximum(m_ref[...], mpart)


def _tc1(q2d, mem, memb, interpret=False):
    return pl.pallas_call(
        _tc1_body,
        grid=(N // TN,),
        in_specs=[
            pl.BlockSpec((TN, D), lambda i: (i, 0)),
            pl.BlockSpec((MS, D), lambda i: (0, 0)),
            pl.BlockSpec((MS, D), lambda i: (0, 0)),
        ],
        out_specs=[
            pl.BlockSpec((TN, D), lambda i: (i, 0)),
            pl.BlockSpec((TN,), lambda i: (i,)),
            pl.BlockSpec((TN,), lambda i: (i,)),
            pl.BlockSpec((MS,), lambda i: (0,)),
        ],
        out_shape=[
            jax.ShapeDtypeStruct((N, D), jnp.float32),
            jax.ShapeDtypeStruct((N,), jnp.int32),
            jax.ShapeDtypeStruct((N,), jnp.float32),
            jax.ShapeDtypeStruct((MS,), jnp.float32),
        ],
        interpret=interpret,
    )(q2d, mem, memb)


def _sc_body(q_hbm, a_hbm, sval_hbm, m_hbm, out_hbm,
             m_v, a_v, sval_v, w_v, qt_v, qt_v2, acc, sem0, sem1):
    cid = lax.axis_index("c")
    sid = lax.axis_index("s")
    pbase = cid * PSC
    f0 = sid * DS

    def zrow(i, carry):
        for k in range(16):
            acc[pl.ds(i * 16 * LANES + k * LANES, LANES)] = (
                jnp.zeros((LANES,), jnp.float32))
        return carry
    lax.fori_loop(0, ACC_N // (16 * LANES), zrow, 0)

    pltpu.sync_copy(m_hbm, m_v)
    pltpu.sync_copy(a_hbm.at[pl.ds(pbase, PSC)], a_v)
    pltpu.sync_copy(sval_hbm.at[pl.ds(pbase, PSC)], sval_v)

    def wbody(i, carry):
        off = i * LANES
        av = a_v[pl.ds(off, LANES)]
        mv = plsc.load_gather(m_v, [av])
        w_v[pl.ds(off, LANES)] = jnp.exp(sval_v[pl.ds(off, LANES)] - mv)
        return carry
    lax.fori_loop(0, PSC // LANES, wbody, 0)

    def _src(ch):
        bidx = cid * (PSC // HW) + ch // (HW // CHP)
        hw0 = (ch % (HW // CHP)) * CHP
        return q_hbm.at[bidx, pl.ds(f0, DS), pl.ds(hw0, CHP)]

    bufs = (qt_v, qt_v2)
    sems = (sem0, sem1)
    cps = [pltpu.async_copy(_src(0), bufs[0], sems[0]), None]
    for ch in range(NCHK):
        cps[ch % 2].wait()
        if ch + 1 < NCHK:
            cps[(ch + 1) % 2] = pltpu.async_copy(
                _src(ch + 1), bufs[(ch + 1) % 2], sems[(ch + 1) % 2])
        buf = bufs[ch % 2]

        def gbody(g, carry):
            goff = ch * CHP + g * LANES
            a16 = a_v[pl.ds(goff, LANES)]
            w16 = w_v[pl.ds(goff, LANES)]
            for fo in range(DS):
                x = buf[fo, pl.ds(g * LANES, LANES)] * w16
                plsc.addupdate_scatter(acc, [a16 + fo * MS], x)
            return carry
        lax.fori_loop(0, CHP // LANES, gbody, 0)

    pltpu.sync_copy(acc, out_hbm.at[cid, sid])


@functools.cache
def _sc_scatter_fn():
    return pl.kernel(
        _sc_body,
        out_type=jax.ShapeDtypeStruct((NC, NS, ACC_N), jnp.float32),
        mesh=plsc.VectorSubcoreMesh(core_axis_name="c", subcore_axis_name="s"),
        scratch_types=[
            pltpu.VMEM((MS,), jnp.float32),
            pltpu.VMEM((PSC,), jnp.int32),
            pltpu.VMEM((PSC,), jnp.float32),
            pltpu.VMEM((PSC,), jnp.float32),
            pltpu.VMEM((DS, CHP), jnp.float32),
            pltpu.VMEM((DS, CHP), jnp.float32),
            pltpu.VMEM((ACC_N,), jnp.float32),
            pltpu.SemaphoreType.DMA,
            pltpu.SemaphoreType.DMA,
        ],
        compiler_params=pltpu.CompilerParams(needs_layout_passes=False),
    )


def _sc_scatter(qr, a, sval, m_seg):
    return _sc_scatter_fn()(qr, a, sval, m_seg)


TMS = 512


def _tc2_body(mem_ref, agg_ref, m_ref, out_ref):
    aggt = jnp.concatenate(
        [agg_ref[0, s] + agg_ref[1, s] for s in range(NS)], axis=0)
    ut = jnp.transpose(mem_ref[...]) + aggt
    nrm = jnp.sqrt(jnp.sum(ut * ut, axis=0, keepdims=True))
    ut = ut / jnp.maximum(nrm, 1e-12)
    ut = jnp.where(m_ref[...] > -1.0e30, ut, 0.0)
    out_ref[...] = jnp.transpose(ut)


def _tc2(mem, agg, m2, interpret=False):
    return pl.pallas_call(
        _tc2_body,
        grid=(MS // TMS,),
        in_specs=[
            pl.BlockSpec((TMS, D), lambda i: (i, 0)),
            pl.BlockSpec((NC, NS, DS, TMS), lambda i: (0, 0, 0, i)),
            pl.BlockSpec((1, TMS), lambda i: (0, i)),
        ],
        out_specs=pl.BlockSpec((TMS, D), lambda i: (i, 0)),
        out_shape=jax.ShapeDtypeStruct((MS, D), jnp.float32),
        interpret=interpret,
    )(mem, agg, m2)


def kernel(query, mem):
    b, d, h, w = query.shape
    q2d = jnp.transpose(query, (0, 2, 3, 1)).reshape(b * h * w, d)
    qh2d, a, sval, m_seg = _tc1(q2d, mem, mem.astype(jnp.bfloat16))
    qr = query.reshape(b, d, h * w)
    agg = _sc_scatter(qr, a, sval, m_seg)
    mem_update = _tc2(mem, agg.reshape(NC, NS, DS, MS), m_seg.reshape(1, MS))
    query_hat = qh2d.reshape(b, h, w, d).transpose(0, 3, 1, 2)
    return (query_hat, mem_update)


# speedup vs baseline: 1.1698x; 1.1698x over previous
import functools

import jax
import jax.numpy as jnp
from jax import lax
from jax.experimental import pallas as pl
from jax.experimental.pallas import tpu as pltpu
from jax.experimental.pallas import tpu_sc as plsc

B = 8
HW = 1024
N = B * HW
D = 512
MS = 2048
TN = 1024
NC, NS = 2, 16
NEG = -3.0e38
LANES = 16

DS = D // NS
PSC = N // NC
CHP = 512
NCHK = PSC // CHP
ACC_N = MS * DS


def _tc1_body(q_ref, mem_ref, memb_ref, qh_ref, a_ref, sval_ref, m_ref):
    i = pl.program_id(0)
    q = q_ref[...]
    score = lax.dot_general(q, mem_ref[...], (((1,), (1,)), ((), ())),
                            preferred_element_type=jnp.float32)
    rowmax = jnp.max(score, axis=1, keepdims=True)
    e = jnp.exp(score - rowmax)
    z = jnp.sum(e, axis=1, keepdims=True)
    cols = lax.broadcasted_iota(jnp.int32, score.shape, 1)
    eqmask = score == rowmax
    a = jnp.min(jnp.where(eqmask, cols, MS), axis=1)
    s_e = jnp.where(e < z * (1.0 / MS), 0.0, e)
    denom = jnp.maximum(jnp.sum(s_e, axis=1, keepdims=True), 1e-12)
    qh = lax.dot_general(s_e.astype(jnp.bfloat16), memb_ref[...],
                         (((1,), (0,)), ((), ())),
                         preferred_element_type=jnp.float32)
    qh_ref[...] = qh * (1.0 / denom)
    a_ref[...] = a
    sval_ref[...] = rowmax[:, 0]
    mpart = jnp.max(jnp.where(eqmask, rowmax, NEG), axis=0)

    @pl.when(i == 0)
    def _():
        m_ref[...] = mpart

    @pl.when(i != 0)
    def _():
        m_ref[...] = jnp.maximum(m_ref[...], mpart)


def _tc1(q2d, mem, memb, interpret=False):
    return pl.pallas_call(
        _tc1_body,
        grid=(N // TN,),
        in_specs=[
            pl.BlockSpec((TN, D), lambda i: (i, 0)),
            pl.BlockSpec((MS, D), lambda i: (0, 0)),
            pl.BlockSpec((MS, D), lambda i: (0, 0)),
        ],
        out_specs=[
            pl.BlockSpec((TN, D), lambda i: (i, 0)),
            pl.BlockSpec((TN,), lambda i: (i,)),
            pl.BlockSpec((TN,), lambda i: (i,)),
            pl.BlockSpec((MS,), lambda i: (0,)),
        ],
        out_shape=[
            jax.ShapeDtypeStruct((N, D), jnp.float32),
            jax.ShapeDtypeStruct((N,), jnp.int32),
            jax.ShapeDtypeStruct((N,), jnp.float32),
            jax.ShapeDtypeStruct((MS,), jnp.float32),
        ],
        interpret=interpret,
    )(q2d, mem, memb)


def _sc_body(q_hbm, a_hbm, sval_hbm, m_hbm, out_hbm,
             m_v, a_v, sval_v, w_v, qt_v, qt_v2, acc, sem0, sem1):
    cid = lax.axis_index("c")
    sid = lax.axis_index("s")
    pbase = cid * PSC
    f0 = sid * DS

    def zrow(i, carry):
        for k in range(16):
            acc[pl.ds(i * 16 * LANES + k * LANES, LANES)] = (
                jnp.zeros((LANES,), jnp.float32))
        return carry
    lax.fori_loop(0, ACC_N // (16 * LANES), zrow, 0)

    pltpu.sync_copy(m_hbm, m_v)
    pltpu.sync_copy(a_hbm.at[pl.ds(pbase, PSC)], a_v)
    pltpu.sync_copy(sval_hbm.at[pl.ds(pbase, PSC)], sval_v)

    def wbody(i, carry):
        off = i * LANES
        av = a_v[pl.ds(off, LANES)]
        mv = plsc.load_gather(m_v, [av])
        w_v[pl.ds(off, LANES)] = jnp.exp(sval_v[pl.ds(off, LANES)] - mv)
        return carry
    lax.fori_loop(0, PSC // LANES, wbody, 0)

    def _src(ch):
        bidx = cid * (PSC // HW) + ch // (HW // CHP)
        hw0 = (ch % (HW // CHP)) * CHP
        return q_hbm.at[bidx, pl.ds(f0, DS), pl.ds(hw0, CHP)]

    bufs = (qt_v, qt_v2)
    sems = (sem0, sem1)
    cps = [pltpu.async_copy(_src(0), bufs[0], sems[0]), None]
    for ch in range(NCHK):
        cps[ch % 2].wait()
        if ch + 1 < NCHK:
            cps[(ch + 1) % 2] = pltpu.async_copy(
                _src(ch + 1), bufs[(ch + 1) % 2], sems[(ch + 1) % 2])
        buf = bufs[ch % 2]

        def gbody(g, carry):
            goff = ch * CHP + g * LANES
            a16 = a_v[pl.ds(goff, LANES)]
            w16 = w_v[pl.ds(goff, LANES)]
            for fo in range(DS):
                x = buf[fo, pl.ds(g * LANES, LANES)] * w16
                plsc.addupdate_scatter(acc.at[pl.ds(fo * MS, MS)], [a16], x)
            return carry
        lax.fori_loop(0, CHP // LANES, gbody, 0)

    pltpu.sync_copy(acc, out_hbm.at[cid, sid])


@functools.cache
def _sc_scatter_fn():
    return pl.kernel(
        _sc_body,
        out_type=jax.ShapeDtypeStruct((NC, NS, ACC_N), jnp.float32),
        mesh=plsc.VectorSubcoreMesh(core_axis_name="c", subcore_axis_name="s"),
        scratch_types=[
            pltpu.VMEM((MS,), jnp.float32),
            pltpu.VMEM((PSC,), jnp.int32),
            pltpu.VMEM((PSC,), jnp.float32),
            pltpu.VMEM((PSC,), jnp.float32),
            pltpu.VMEM((DS, CHP), jnp.float32),
            pltpu.VMEM((DS, CHP), jnp.float32),
            pltpu.VMEM((ACC_N,), jnp.float32),
            pltpu.SemaphoreType.DMA,
            pltpu.SemaphoreType.DMA,
        ],
        compiler_params=pltpu.CompilerParams(needs_layout_passes=False),
    )


def _sc_scatter(qr, a, sval, m_seg):
    return _sc_scatter_fn()(qr, a, sval, m_seg)


TMS = 512


def _tc2_body(mem_ref, agg_ref, m_ref, out_ref):
    aggt = jnp.concatenate(
        [agg_ref[0, s] + agg_ref[1, s] for s in range(NS)], axis=0)
    ut = jnp.transpose(mem_ref[...]) + aggt
    nrm = jnp.sqrt(jnp.sum(ut * ut, axis=0, keepdims=True))
    ut = ut / jnp.maximum(nrm, 1e-12)
    ut = jnp.where(m_ref[...] > -1.0e30, ut, 0.0)
    out_ref[...] = jnp.transpose(ut)


def _tc2(mem, agg, m2, interpret=False):
    return pl.pallas_call(
        _tc2_body,
        grid=(MS // TMS,),
        in_specs=[
            pl.BlockSpec((TMS, D), lambda i: (i, 0)),
            pl.BlockSpec((NC, NS, DS, TMS), lambda i: (0, 0, 0, i)),
            pl.BlockSpec((1, TMS), lambda i: (0, i)),
        ],
        out_specs=pl.BlockSpec((TMS, D), lambda i: (i, 0)),
        out_shape=jax.ShapeDtypeStruct((MS, D), jnp.float32),
        interpret=interpret,
    )(mem, agg, m2)


def kernel(query, mem):
    b, d, h, w = query.shape
    q2d = jnp.transpose(query, (0, 2, 3, 1)).reshape(b * h * w, d)
    qh2d, a, sval, m_seg = _tc1(q2d, mem, mem.astype(jnp.bfloat16))
    qr = query.reshape(b, d, h * w)
    agg = _sc_scatter(qr, a, sval, m_seg)
    mem_update = _tc2(mem, agg.reshape(NC, NS, DS, MS), m_seg.reshape(1, MS))
    query_hat = qh2d.reshape(b, h, w, d).transpose(0, 3, 1, 2)
    return (query_hat, mem_update)


# speedup vs baseline: 1.1839x; 1.0120x over previous
import functools

import jax
import jax.numpy as jnp
from jax import lax
from jax.experimental import pallas as pl
from jax.experimental.pallas import tpu as pltpu
from jax.experimental.pallas import tpu_sc as plsc

B = 8
HW = 1024
N = B * HW
D = 512
MS = 2048
TN = 1024
NC, NS = 2, 16
NEG = -3.0e38
LANES = 16

DS = D // NS
PSC = N // NC
CHP = 512
NCHK = PSC // CHP
ACC_N = MS * DS


def _tc1_body(q_ref, mem_ref, memb_ref, qh_ref, a_ref, sval_ref, m_ref):
    i = pl.program_id(0)
    q = q_ref[...]
    score = lax.dot_general(q, mem_ref[...], (((1,), (1,)), ((), ())),
                            preferred_element_type=jnp.float32)
    rowmax = jnp.max(score, axis=1, keepdims=True)
    e = jnp.exp(score - rowmax)
    z = jnp.sum(e, axis=1, keepdims=True)
    cols = lax.broadcasted_iota(jnp.int32, score.shape, 1).astype(jnp.float32)
    eqmask = score == rowmax
    a = jnp.min(jnp.where(eqmask, cols, float(MS)), axis=1).astype(jnp.int32)
    s_e = jnp.where(e < z * (1.0 / MS), 0.0, e)
    denom = jnp.maximum(jnp.sum(s_e, axis=1, keepdims=True), 1e-12)
    qh = lax.dot_general(s_e.astype(jnp.bfloat16), memb_ref[...],
                         (((1,), (0,)), ((), ())),
                         preferred_element_type=jnp.float32)
    qh_ref[...] = qh * (1.0 / denom)
    a_ref[...] = a
    sval_ref[...] = rowmax[:, 0]
    mpart = jnp.max(jnp.where(eqmask, rowmax, NEG), axis=0)

    @pl.when(i == 0)
    def _():
        m_ref[...] = mpart

    @pl.when(i != 0)
    def _():
        m_ref[...] = jnp.maximum(m_ref[...], mpart)


def _tc1(q2d, mem, memb, interpret=False):
    return pl.pallas_call(
        _tc1_body,
        grid=(N // TN,),
        in_specs=[
            pl.BlockSpec((TN, D), lambda i: (i, 0)),
            pl.BlockSpec((MS, D), lambda i: (0, 0)),
            pl.BlockSpec((MS, D), lambda i: (0, 0)),
        ],
        out_specs=[
            pl.BlockSpec((TN, D), lambda i: (i, 0)),
            pl.BlockSpec((TN,), lambda i: (i,)),
            pl.BlockSpec((TN,), lambda i: (i,)),
            pl.BlockSpec((MS,), lambda i: (0,)),
        ],
        out_shape=[
            jax.ShapeDtypeStruct((N, D), jnp.float32),
            jax.ShapeDtypeStruct((N,), jnp.int32),
            jax.ShapeDtypeStruct((N,), jnp.float32),
            jax.ShapeDtypeStruct((MS,), jnp.float32),
        ],
        interpret=interpret,
    )(q2d, mem, memb)


def _sc_body(q_hbm, a_hbm, sval_hbm, m_hbm, out_hbm,
             m_v, a_v, sval_v, w_v, qt_v, qt_v2, acc, sem0, sem1):
    cid = lax.axis_index("c")
    sid = lax.axis_index("s")
    pbase = cid * PSC
    f0 = sid * DS

    def zrow(i, carry):
        for k in range(16):
            acc[pl.ds(i * 16 * LANES + k * LANES, LANES)] = (
                jnp.zeros((LANES,), jnp.float32))
        return carry
    lax.fori_loop(0, ACC_N // (16 * LANES), zrow, 0)

    pltpu.sync_copy(m_hbm, m_v)
    pltpu.sync_copy(a_hbm.at[pl.ds(pbase, PSC)], a_v)
    pltpu.sync_copy(sval_hbm.at[pl.ds(pbase, PSC)], sval_v)

    def wbody(i, carry):
        off = i * LANES
        av = a_v[pl.ds(off, LANES)]
        mv = plsc.load_gather(m_v, [av])
        w_v[pl.ds(off, LANES)] = jnp.exp(sval_v[pl.ds(off, LANES)] - mv)
        return carry
    lax.fori_loop(0, PSC // LANES, wbody, 0)

    def _src(ch):
        bidx = cid * (PSC // HW) + ch // (HW // CHP)
        hw0 = (ch % (HW // CHP)) * CHP
        return q_hbm.at[bidx, pl.ds(f0, DS), pl.ds(hw0, CHP)]

    bufs = (qt_v, qt_v2)
    sems = (sem0, sem1)
    cps = [pltpu.async_copy(_src(0), bufs[0], sems[0]), None]
    for ch in range(NCHK):
        cps[ch % 2].wait()
        if ch + 1 < NCHK:
            cps[(ch + 1) % 2] = pltpu.async_copy(
                _src(ch + 1), bufs[(ch + 1) % 2], sems[(ch + 1) % 2])
        buf = bufs[ch % 2]

        def gbody(g, carry):
            goff = ch * CHP + g * LANES
            a16 = a_v[pl.ds(goff, LANES)]
            w16 = w_v[pl.ds(goff, LANES)]
            for fo in range(DS):
                x = buf[fo, pl.ds(g * LANES, LANES)] * w16
                plsc.addupdate_scatter(acc.at[pl.ds(fo * MS, MS)], [a16], x)
            return carry
        lax.fori_loop(0, CHP // LANES, gbody, 0)

    pltpu.sync_copy(acc, out_hbm.at[cid, sid])


@functools.cache
def _sc_scatter_fn():
    return pl.kernel(
        _sc_body,
        out_type=jax.ShapeDtypeStruct((NC, NS, ACC_N), jnp.float32),
        mesh=plsc.VectorSubcoreMesh(core_axis_name="c", subcore_axis_name="s"),
        scratch_types=[
            pltpu.VMEM((MS,), jnp.float32),
            pltpu.VMEM((PSC,), jnp.int32),
            pltpu.VMEM((PSC,), jnp.float32),
            pltpu.VMEM((PSC,), jnp.float32),
            pltpu.VMEM((DS, CHP), jnp.float32),
            pltpu.VMEM((DS, CHP), jnp.float32),
            pltpu.VMEM((ACC_N,), jnp.float32),
            pltpu.SemaphoreType.DMA,
            pltpu.SemaphoreType.DMA,
        ],
        compiler_params=pltpu.CompilerParams(needs_layout_passes=False),
    )


def _sc_scatter(qr, a, sval, m_seg):
    return _sc_scatter_fn()(qr, a, sval, m_seg)


TMS = 512


def _tc2_body(mem_ref, agg_ref, m_ref, out_ref):
    aggt = jnp.concatenate(
        [agg_ref[0, s] + agg_ref[1, s] for s in range(NS)], axis=0)
    ut = jnp.transpose(mem_ref[...]) + aggt
    nrm = jnp.sqrt(jnp.sum(ut * ut, axis=0, keepdims=True))
    ut = ut / jnp.maximum(nrm, 1e-12)
    ut = jnp.where(m_ref[...] > -1.0e30, ut, 0.0)
    out_ref[...] = jnp.transpose(ut)


def _tc2(mem, agg, m2, interpret=False):
    return pl.pallas_call(
        _tc2_body,
        grid=(MS // TMS,),
        in_specs=[
            pl.BlockSpec((TMS, D), lambda i: (i, 0)),
            pl.BlockSpec((NC, NS, DS, TMS), lambda i: (0, 0, 0, i)),
            pl.BlockSpec((1, TMS), lambda i: (0, i)),
        ],
        out_specs=pl.BlockSpec((TMS, D), lambda i: (i, 0)),
        out_shape=jax.ShapeDtypeStruct((MS, D), jnp.float32),
        interpret=interpret,
    )(mem, agg, m2)


def kernel(query, mem):
    b, d, h, w = query.shape
    q2d = jnp.transpose(query, (0, 2, 3, 1)).reshape(b * h * w, d)
    qh2d, a, sval, m_seg = _tc1(q2d, mem, mem.astype(jnp.bfloat16))
    qr = query.reshape(b, d, h * w)
    agg = _sc_scatter(qr, a, sval, m_seg)
    mem_update = _tc2(mem, agg.reshape(NC, NS, DS, MS), m_seg.reshape(1, MS))
    query_hat = qh2d.reshape(b, h, w, d).transpose(0, 3, 1, 2)
    return (query_hat, mem_update)


# speedup vs baseline: 1.2033x; 1.0164x over previous
import functools

import jax
import jax.numpy as jnp
from jax import lax
from jax.experimental import pallas as pl
from jax.experimental.pallas import tpu as pltpu
from jax.experimental.pallas import tpu_sc as plsc

B = 8
HW = 1024
N = B * HW
D = 512
MS = 2048
TN = 1024
NC, NS = 2, 16
NEG = -3.0e38
LANES = 16

DS = D // NS
PSC = N // NC
CHP = 512
NCHK = PSC // CHP
ACC_N = MS * DS


def _tc1_body(q_ref, mem_ref, memb_ref, qh_ref, a_ref, sval_ref, m_ref):
    i = pl.program_id(0)
    q = q_ref[...]
    score = lax.dot_general(q, mem_ref[...], (((1,), (1,)), ((), ())),
                            preferred_element_type=jnp.float32)
    rowmax = jnp.max(score, axis=1, keepdims=True)
    e = jnp.exp(score - rowmax)
    z = jnp.sum(e, axis=1, keepdims=True)
    cols = lax.broadcasted_iota(jnp.int32, score.shape, 1).astype(jnp.float32)
    eqmask = score == rowmax
    a = jnp.min(jnp.where(eqmask, cols, float(MS)), axis=1).astype(jnp.int32)
    s_e = jnp.where(e < z * (1.0 / MS), 0.0, e)
    denom = jnp.maximum(jnp.sum(s_e, axis=1, keepdims=True), 1e-12)
    qh = lax.dot_general(s_e.astype(jnp.bfloat16), memb_ref[...],
                         (((1,), (0,)), ((), ())),
                         preferred_element_type=jnp.float32)
    qh_ref[...] = qh * (1.0 / denom)
    a_ref[...] = a
    sval_ref[...] = rowmax[:, 0]
    mpart = jnp.max(jnp.where(eqmask, rowmax, NEG), axis=0)

    @pl.when(i == 0)
    def _():
        m_ref[...] = mpart

    @pl.when(i != 0)
    def _():
        m_ref[...] = jnp.maximum(m_ref[...], mpart)


def _tc1(q2d, mem, memb, interpret=False):
    return pl.pallas_call(
        _tc1_body,
        grid=(N // TN,),
        in_specs=[
            pl.BlockSpec((TN, D), lambda i: (i, 0)),
            pl.BlockSpec((MS, D), lambda i: (0, 0)),
            pl.BlockSpec((MS, D), lambda i: (0, 0)),
        ],
        out_specs=[
            pl.BlockSpec((TN, D), lambda i: (i, 0)),
            pl.BlockSpec((TN,), lambda i: (i,)),
            pl.BlockSpec((TN,), lambda i: (i,)),
            pl.BlockSpec((MS,), lambda i: (0,)),
        ],
        out_shape=[
            jax.ShapeDtypeStruct((N, D), jnp.float32),
            jax.ShapeDtypeStruct((N,), jnp.int32),
            jax.ShapeDtypeStruct((N,), jnp.float32),
            jax.ShapeDtypeStruct((MS,), jnp.float32),
        ],
        interpret=interpret,
    )(q2d, mem, memb)


def _sc_body(q_hbm, a_hbm, sval_hbm, m_hbm, out_hbm,
             m_v, a_v, sval_v, w_v, qt_v, qt_v2, acc, sem0, sem1):
    cid = lax.axis_index("c")
    sid = lax.axis_index("s")
    pbase = cid * PSC
    f0 = sid * DS

    mcp = pltpu.async_copy(m_hbm, m_v, sem0)
    acp = pltpu.async_copy(a_hbm.at[pl.ds(pbase, PSC)], a_v, sem1)
    scp = pltpu.async_copy(sval_hbm.at[pl.ds(pbase, PSC)], sval_v, sem0)

    def zrow(i, carry):
        for k in range(16):
            acc[pl.ds(i * 16 * LANES + k * LANES, LANES)] = (
                jnp.zeros((LANES,), jnp.float32))
        return carry
    lax.fori_loop(0, ACC_N // (16 * LANES), zrow, 0)
    mcp.wait()
    acp.wait()
    scp.wait()

    def wbody(i, carry):
        off = i * LANES
        av = a_v[pl.ds(off, LANES)]
        mv = plsc.load_gather(m_v, [av])
        w_v[pl.ds(off, LANES)] = jnp.exp(sval_v[pl.ds(off, LANES)] - mv)
        return carry
    lax.fori_loop(0, PSC // LANES, wbody, 0)

    def _src(ch):
        bidx = cid * (PSC // HW) + ch // (HW // CHP)
        hw0 = (ch % (HW // CHP)) * CHP
        return q_hbm.at[bidx, pl.ds(f0, DS), pl.ds(hw0, CHP)]

    bufs = (qt_v, qt_v2)
    sems = (sem0, sem1)
    cps = [pltpu.async_copy(_src(0), bufs[0], sems[0]), None]
    for ch in range(NCHK):
        cps[ch % 2].wait()
        if ch + 1 < NCHK:
            cps[(ch + 1) % 2] = pltpu.async_copy(
                _src(ch + 1), bufs[(ch + 1) % 2], sems[(ch + 1) % 2])
        buf = bufs[ch % 2]

        def gbody(g, carry):
            goff = ch * CHP + g * LANES
            a16 = a_v[pl.ds(goff, LANES)]
            w16 = w_v[pl.ds(goff, LANES)]
            for fo in range(DS):
                x = buf[fo, pl.ds(g * LANES, LANES)] * w16
                plsc.addupdate_scatter(acc.at[pl.ds(fo * MS, MS)], [a16], x)
            return carry
        lax.fori_loop(0, CHP // LANES, gbody, 0)

    pltpu.sync_copy(acc, out_hbm.at[cid, sid])


@functools.cache
def _sc_scatter_fn():
    return pl.kernel(
        _sc_body,
        out_type=jax.ShapeDtypeStruct((NC, NS, ACC_N), jnp.float32),
        mesh=plsc.VectorSubcoreMesh(core_axis_name="c", subcore_axis_name="s"),
        scratch_types=[
            pltpu.VMEM((MS,), jnp.float32),
            pltpu.VMEM((PSC,), jnp.int32),
            pltpu.VMEM((PSC,), jnp.float32),
            pltpu.VMEM((PSC,), jnp.float32),
            pltpu.VMEM((DS, CHP), jnp.float32),
            pltpu.VMEM((DS, CHP), jnp.float32),
            pltpu.VMEM((ACC_N,), jnp.float32),
            pltpu.SemaphoreType.DMA,
            pltpu.SemaphoreType.DMA,
        ],
        compiler_params=pltpu.CompilerParams(needs_layout_passes=False),
    )


def _sc_scatter(qr, a, sval, m_seg):
    return _sc_scatter_fn()(qr, a, sval, m_seg)


TMS = 512


def _tc2_body(mem_ref, agg_ref, m_ref, out_ref):
    aggt = jnp.concatenate(
        [agg_ref[0, s] + agg_ref[1, s] for s in range(NS)], axis=0)
    ut = jnp.transpose(mem_ref[...]) + aggt
    nrm = jnp.sqrt(jnp.sum(ut * ut, axis=0, keepdims=True))
    ut = ut / jnp.maximum(nrm, 1e-12)
    ut = jnp.where(m_ref[...] > -1.0e30, ut, 0.0)
    out_ref[...] = jnp.transpose(ut)


def _tc2(mem, agg, m2, interpret=False):
    return pl.pallas_call(
        _tc2_body,
        grid=(MS // TMS,),
        in_specs=[
            pl.BlockSpec((TMS, D), lambda i: (i, 0)),
            pl.BlockSpec((NC, NS, DS, TMS), lambda i: (0, 0, 0, i)),
            pl.BlockSpec((1, TMS), lambda i: (0, i)),
        ],
        out_specs=pl.BlockSpec((TMS, D), lambda i: (i, 0)),
        out_shape=jax.ShapeDtypeStruct((MS, D), jnp.float32),
        interpret=interpret,
    )(mem, agg, m2)


def kernel(query, mem):
    b, d, h, w = query.shape
    q2d = jnp.transpose(query, (0, 2, 3, 1)).reshape(b * h * w, d)
    qh2d, a, sval, m_seg = _tc1(q2d, mem, mem.astype(jnp.bfloat16))
    qr = query.reshape(b, d, h * w)
    agg = _sc_scatter(qr, a, sval, m_seg)
    mem_update = _tc2(mem, agg.reshape(NC, NS, DS, MS), m_seg.reshape(1, MS))
    query_hat = qh2d.reshape(b, h, w, d).transpose(0, 3, 1, 2)
    return (query_hat, mem_update)


# speedup vs baseline: 1.2077x; 1.0037x over previous
import functools

import jax
import jax.numpy as jnp
from jax import lax
from jax.experimental import pallas as pl
from jax.experimental.pallas import tpu as pltpu
from jax.experimental.pallas import tpu_sc as plsc

B = 8
HW = 1024
N = B * HW
D = 512
MS = 2048
TN = 1024
NC, NS = 2, 16
NEG = -3.0e38
LANES = 16

DS = D // NS
PSC = N // NC
CHP = 512
NCHK = PSC // CHP
ACC_N = MS * DS


def _tc1_body(q_ref, mem_ref, memb_ref, qh_ref, a_ref, sval_ref, m_ref):
    i = pl.program_id(0)
    q = q_ref[...]
    score = lax.dot_general(q, mem_ref[...], (((1,), (1,)), ((), ())),
                            preferred_element_type=jnp.float32)
    rowmax = jnp.max(score, axis=1, keepdims=True)
    e = jnp.exp(score - rowmax)
    z = jnp.sum(e, axis=1, keepdims=True)
    cols = lax.broadcasted_iota(jnp.int32, score.shape, 1).astype(jnp.float32)
    eqmask = score == rowmax
    a = jnp.min(jnp.where(eqmask, cols, float(MS)), axis=1).astype(jnp.int32)
    s_e = jnp.where(e < z * (1.0 / MS), 0.0, e)
    denom = jnp.maximum(jnp.sum(s_e, axis=1, keepdims=True), 1e-12)
    qh = lax.dot_general(s_e.astype(jnp.bfloat16), memb_ref[...],
                         (((1,), (0,)), ((), ())),
                         preferred_element_type=jnp.float32)
    qh_ref[...] = qh * (1.0 / denom)
    a_ref[...] = a
    sval_ref[...] = rowmax[:, 0]
    mpart = jnp.max(jnp.where(eqmask, rowmax, NEG), axis=0)

    @pl.when(i == 0)
    def _():
        m_ref[...] = mpart

    @pl.when(i != 0)
    def _():
        m_ref[...] = jnp.maximum(m_ref[...], mpart)


def _tc1(q2d, mem, memb, interpret=False):
    return pl.pallas_call(
        _tc1_body,
        grid=(N // TN,),
        in_specs=[
            pl.BlockSpec((TN, D), lambda i: (i, 0)),
            pl.BlockSpec((MS, D), lambda i: (0, 0)),
            pl.BlockSpec((MS, D), lambda i: (0, 0)),
        ],
        out_specs=[
            pl.BlockSpec((TN, D), lambda i: (i, 0)),
            pl.BlockSpec((TN,), lambda i: (i,)),
            pl.BlockSpec((TN,), lambda i: (i,)),
            pl.BlockSpec((MS,), lambda i: (0,)),
        ],
        out_shape=[
            jax.ShapeDtypeStruct((N, D), jnp.float32),
            jax.ShapeDtypeStruct((N,), jnp.int32),
            jax.ShapeDtypeStruct((N,), jnp.float32),
            jax.ShapeDtypeStruct((MS,), jnp.float32),
        ],
        interpret=interpret,
    )(q2d, mem, memb)


def _sc_body(q_hbm, a_hbm, sval_hbm, m_hbm, out_hbm,
             m_v, a_v, sval_v, w_v, qt_v, qt_v2, acc,
             semm, sema, semsv, semq0, semq1):
    cid = lax.axis_index("c")
    sid = lax.axis_index("s")
    pbase = cid * PSC
    f0 = sid * DS

    def _src(ch):
        bidx = cid * (PSC // HW) + ch // (HW // CHP)
        hw0 = (ch % (HW // CHP)) * CHP
        return q_hbm.at[bidx, pl.ds(f0, DS), pl.ds(hw0, CHP)]

    mcp = pltpu.async_copy(m_hbm, m_v, semm)
    acp = pltpu.async_copy(a_hbm.at[pl.ds(pbase, PSC)], a_v, sema)
    scp = pltpu.async_copy(sval_hbm.at[pl.ds(pbase, PSC)], sval_v, semsv)
    bufs = (qt_v, qt_v2)
    sems = (semq0, semq1)
    cps = [pltpu.async_copy(_src(0), bufs[0], sems[0]),
           pltpu.async_copy(_src(1), bufs[1], sems[1])]

    def zrow(i, carry):
        for k in range(16):
            acc[pl.ds(i * 16 * LANES + k * LANES, LANES)] = (
                jnp.zeros((LANES,), jnp.float32))
        return carry
    lax.fori_loop(0, ACC_N // (16 * LANES), zrow, 0)
    mcp.wait()
    acp.wait()
    scp.wait()

    def wbody(i, carry):
        off = i * LANES
        av = a_v[pl.ds(off, LANES)]
        mv = plsc.load_gather(m_v, [av])
        w_v[pl.ds(off, LANES)] = jnp.exp(sval_v[pl.ds(off, LANES)] - mv)
        return carry
    lax.fori_loop(0, PSC // LANES, wbody, 0)

    for ch in range(NCHK):
        cps[ch % 2].wait()
        buf = bufs[ch % 2]

        def gbody(g, carry):
            goff = ch * CHP + g * LANES
            a16 = a_v[pl.ds(goff, LANES)]
            w16 = w_v[pl.ds(goff, LANES)]
            for fo in range(DS):
                x = buf[fo, pl.ds(g * LANES, LANES)] * w16
                plsc.addupdate_scatter(acc.at[pl.ds(fo * MS, MS)], [a16], x)
            return carry
        lax.fori_loop(0, CHP // LANES, gbody, 0)
        if ch + 2 < NCHK:
            cps[ch % 2] = pltpu.async_copy(
                _src(ch + 2), bufs[ch % 2], sems[ch % 2])

    pltpu.sync_copy(acc, out_hbm.at[cid, sid])


@functools.cache
def _sc_scatter_fn():
    return pl.kernel(
        _sc_body,
        out_type=jax.ShapeDtypeStruct((NC, NS, ACC_N), jnp.float32),
        mesh=plsc.VectorSubcoreMesh(core_axis_name="c", subcore_axis_name="s"),
        scratch_types=[
            pltpu.VMEM((MS,), jnp.float32),
            pltpu.VMEM((PSC,), jnp.int32),
            pltpu.VMEM((PSC,), jnp.float32),
            pltpu.VMEM((PSC,), jnp.float32),
            pltpu.VMEM((DS, CHP), jnp.float32),
            pltpu.VMEM((DS, CHP), jnp.float32),
            pltpu.VMEM((ACC_N,), jnp.float32),
            pltpu.SemaphoreType.DMA,
            pltpu.SemaphoreType.DMA,
            pltpu.SemaphoreType.DMA,
            pltpu.SemaphoreType.DMA,
            pltpu.SemaphoreType.DMA,
        ],
        compiler_params=pltpu.CompilerParams(needs_layout_passes=False),
    )


def _sc_scatter(qr, a, sval, m_seg):
    return _sc_scatter_fn()(qr, a, sval, m_seg)


TMS = 512


def _tc2_body(mem_ref, agg_ref, m_ref, out_ref):
    aggt = jnp.concatenate(
        [agg_ref[0, s] + agg_ref[1, s] for s in range(NS)], axis=0)
    ut = jnp.transpose(mem_ref[...]) + aggt
    nrm = jnp.sqrt(jnp.sum(ut * ut, axis=0, keepdims=True))
    ut = ut / jnp.maximum(nrm, 1e-12)
    ut = jnp.where(m_ref[...] > -1.0e30, ut, 0.0)
    out_ref[...] = jnp.transpose(ut)


def _tc2(mem, agg, m2, interpret=False):
    return pl.pallas_call(
        _tc2_body,
        grid=(MS // TMS,),
        in_specs=[
            pl.BlockSpec((TMS, D), lambda i: (i, 0)),
            pl.BlockSpec((NC, NS, DS, TMS), lambda i: (0, 0, 0, i)),
            pl.BlockSpec((1, TMS), lambda i: (0, i)),
        ],
        out_specs=pl.BlockSpec((TMS, D), lambda i: (i, 0)),
        out_shape=jax.ShapeDtypeStruct((MS, D), jnp.float32),
        interpret=interpret,
    )(mem, agg, m2)


def kernel(query, mem):
    b, d, h, w = query.shape
    q2d = jnp.transpose(query, (0, 2, 3, 1)).reshape(b * h * w, d)
    qh2d, a, sval, m_seg = _tc1(q2d, mem, mem.astype(jnp.bfloat16))
    qr = query.reshape(b, d, h * w)
    agg = _sc_scatter(qr, a, sval, m_seg)
    mem_update = _tc2(mem, agg.reshape(NC, NS, DS, MS), m_seg.reshape(1, MS))
    query_hat = qh2d.reshape(b, h, w, d).transpose(0, 3, 1, 2)
    return (query_hat, mem_update)
